# baseline (speedup 1.0000x reference)
import functools

import jax
import jax.numpy as jnp
from jax import lax
from jax.experimental import pallas as pl
from jax.experimental.pallas import tpu as pltpu
from jax.experimental.pallas import tpu_sc as plsc

NTYPES = ["drug", "protein", "pathway", "side_effect"]
NNODES = {"drug": 10000, "protein": 50000, "pathway": 10000, "side_effect": 10000}
ETYPES = [
    ("drug", "treats", "side_effect"),
    ("drug", "targets", "protein"),
    ("protein", "in_pathway", "pathway"),
    ("protein", "causes", "side_effect"),
    ("side_effect", "rev_treats", "drug"),
    ("protein", "rev_targets", "drug"),
    ("pathway", "rev_in_pathway", "protein"),
]
D_IN, HID = 128, 256
E = 80000

NS = 16
T = 128
NT = 40
EP = NS * T * NT
NBUF = 4

SC_PLAN = [
    (32, 4, ((("targets", None),), (("rev_in_pathway", None),))),
    (128, 2, ((("treats", None), ("rev_treats", None), ("in_pathway", 0)),
              (("causes", None), ("rev_targets", None), ("in_pathway", 1)))),
]
EDGE_BY_REL = {rel: (s, d) for (s, rel, d) in ETYPES}
RELS = [rel for (_, rel, _) in ETYPES]
CHUNK32_SRCS = ("drug", "pathway")


def _pad128(n):
    return ((n + 128) // 128) * 128


ZROWS = _pad128(50000) // NS

_MESH = plsc.VectorSubcoreMesh(core_axis_name="c", subcore_axis_name="s")
_SC_PARAMS = pltpu.CompilerParams(use_tc_tiling_on_sc=False)


def _agg_type(cw, nbuf, nch, n_dst, nt_rows, row0, tables, src_hbm,
              dst_hbm, zbuf, out_hbm, acc, src_v, dst_v, rowbufs, gsems, sid):
    n_pad = _pad128(n_dst)
    rows_t = n_pad // NS
    zr = 4096 // cw
    base = row0 + sid * nt_rows
    pltpu.sync_copy(src_hbm.at[pl.ds(base, nt_rows)],
                    src_v.at[pl.ds(0, nt_rows)])
    pltpu.sync_copy(dst_hbm.at[pl.ds(base, nt_rows)],
                    dst_v.at[pl.ds(0, nt_rows)])
    for ch in range(nch):
        tcol = tables[ch]
        zoff = sid * rows_t

        def zstep(k, carry):
            pltpu.sync_copy(zbuf, acc.at[pl.ds(zoff + k * zr, zr)])
            return carry

        lax.fori_loop(0, rows_t // zr, zstep, 0)
        rem = rows_t % zr
        if rem:
            pltpu.sync_copy(
                zbuf.at[pl.ds(0, rem)],
                acc.at[pl.ds(zoff + (rows_t // zr) * zr, rem)])
        plsc.subcore_barrier()

        for b in range(nbuf):
            pltpu.async_copy(tcol.at[src_v.at[b]], rowbufs.at[b], gsems[b])

        def outer(g, carry, tcol=tcol):
            for b in range(nbuf):
                j = g * nbuf + b
                pltpu.make_async_copy(
                    tcol.at[src_v.at[j]], rowbufs.at[b], gsems[b]).wait()
                pltpu.sync_copy(rowbufs.at[b], acc.at[dst_v.at[j]], add=True)

                @pl.when(j + nbuf < nt_rows)
                def _():
                    pltpu.async_copy(tcol.at[src_v.at[j + nbuf]],
                                     rowbufs.at[b], gsems[b])
            return carry

        lax.fori_loop(0, nt_rows // nbuf, outer, 0)
        plsc.subcore_barrier()
        pltpu.sync_copy(acc.at[pl.ds(sid * rows_t, rows_t)],
                        out_hbm.at[pl.ds(sid * rows_t, rows_t),
                                   pl.ds(ch * cw, cw)])
        plsc.subcore_barrier()


@functools.lru_cache(maxsize=None)
def _make_agg_layer(d_in, cw, nbuf, work):
    nch = d_in // cw
    ents = [e for core in work for e in core]
    n_outs = [NNODES[EDGE_BY_REL[rel][1]] for (rel, _) in ents]
    acc_rows = _pad128(max(n_outs))

    def body(*refs):
        p = 0
        tabs, eidx = [], []
        for _ in ents:
            tabs.append(refs[p:p + nch])
            eidx.append((refs[p + nch], refs[p + nch + 1]))
            p += nch + 2
        nout = len(ents)
        outs = refs[p:p + nout]
        p += nout
        acc = refs[p]
        src_v, dst_v, rowbufs, zbuf = refs[p + 1:p + 5]
        gsems = refs[p + 5:]
        cid = lax.axis_index("c")
        sid = lax.axis_index("s")

        def zfill(i, carry):
            for w in range(cw // 16):
                zbuf[i, pl.ds(w * 16, 16)] = jnp.zeros((16,), jnp.float32)
            return carry

        lax.fori_loop(0, 4096 // cw, zfill, 0)

        def run(core):
            oi = 0 if core == 0 else len(work[0])
            for (rel, half) in work[core]:
                if half is None:
                    nt_rows, row0 = NT, 0
                else:
                    nt_rows, row0 = NT // 2, half * (EP // T // 2)
                _agg_type(cw, nbuf, nch, n_outs[oi], nt_rows, row0, tabs[oi],
                          eidx[oi][0], eidx[oi][1], zbuf, outs[oi],
                          acc, src_v, dst_v, rowbufs, gsems, sid)
                oi += 1

        @pl.when(cid == 0)
        def _():
            run(0)

        @pl.when(cid == 1)
        def _():
            run(1)

    return pl.kernel(
        body,
        out_type=[jax.ShapeDtypeStruct((_pad128(n), d_in), jnp.float32)
                  for n in n_outs],
        mesh=_MESH,
        compiler_params=_SC_PARAMS,
        scratch_types=[
            pltpu.VMEM_SHARED((acc_rows, cw), jnp.float32),
            pltpu.VMEM((NT, T), jnp.int32),
            pltpu.VMEM((NT, T), jnp.int32),
            pltpu.VMEM((nbuf, T, cw), jnp.float32),
            pltpu.VMEM((4096 // cw, cw), jnp.float32),
        ] + [pltpu.SemaphoreType.DMA] * nbuf,
    )


@functools.lru_cache(maxsize=None)
def _make_counts():
    n_dsts = tuple(NNODES[d] for (_, _, d) in ETYPES)

    def body(*refs):
        dsts = refs[:7]
        ones_hbm, zeros_hbm = refs[7:9]
        outs = refs[9:16]
        acc, dst_v, onesbuf = refs[16:]
        cid = lax.axis_index("c")
        sid = lax.axis_index("s")
        base = sid * NT
        pltpu.sync_copy(ones_hbm, onesbuf)

        def one_type(t):
            rows_t = _pad128(n_dsts[t]) // NS
            pltpu.sync_copy(zeros_hbm.at[pl.ds(0, rows_t)],
                            acc.at[pl.ds(sid * rows_t, rows_t)])
            pltpu.sync_copy(dsts[t].at[pl.ds(base, NT)], dst_v)
            plsc.subcore_barrier()

            def step(j, carry):
                pltpu.sync_copy(onesbuf, acc.at[dst_v.at[j]], add=True)
                return carry

            lax.fori_loop(0, NT, step, 0)
            plsc.subcore_barrier()
            pltpu.sync_copy(acc.at[pl.ds(sid * rows_t, rows_t)],
                            outs[t].at[pl.ds(sid * rows_t, rows_t)])
            plsc.subcore_barrier()

        @pl.when(cid == 0)
        def _():
            for t in (0, 1, 2, 3):
                one_type(t)

        @pl.when(cid == 1)
        def _():
            for t in (4, 5, 6):
                one_type(t)

    return pl.kernel(
        body,
        out_type=[jax.ShapeDtypeStruct((_pad128(n), 16), jnp.float32)
                  for n in n_dsts],
        mesh=_MESH,
        compiler_params=_SC_PARAMS,
        scratch_types=[
            pltpu.VMEM_SHARED((_pad128(50000), 16), jnp.float32),
            pltpu.VMEM((NT, T), jnp.int32),
            pltpu.VMEM((T, 16), jnp.float32),
        ],
    )


def _z_body(n_rels, npc, nblk, *refs):
    i = pl.program_id(0)
    aggs = refs[0:n_rels]
    cnts = refs[n_rels:2 * n_rels]
    wls = refs[2 * n_rels:3 * n_rels]
    xps = refs[3 * n_rels:3 * n_rels + npc]
    wr_ref, bias_ref, z_ref, st_ref = refs[3 * n_rels + npc:]
    z = bias_ref[...]
    for p in range(npc):
        z = z + jnp.dot(xps[p][...], wr_ref[pl.ds(p * 128, 128), :],
                        preferred_element_type=jnp.float32)
    for r in range(n_rels):
        a = aggs[r][...]
        cnt = cnts[r][:, 0:1]
        mean = a * (1.0 / jnp.maximum(cnt, 1.0))
        z = z + jnp.dot(mean, wls[r][...], preferred_element_type=jnp.float32)
    z_ref[...] = z

    @pl.when(i == 0)
    def _():
        st_ref[...] = jnp.zeros_like(st_ref)

    st_ref[0:1, :] += jnp.sum(z, axis=0, keepdims=True)
    st_ref[1:2, :] += jnp.sum(z * z, axis=0, keepdims=True)


@functools.lru_cache(maxsize=None)
def _make_z(n, d_in, n_rels, blk):
    nblk = n // blk
    npc = d_in // 128
    in_specs = (
        [pl.BlockSpec((blk, d_in), lambda i: (i, 0))
         for _ in range(n_rels)]
        + [pl.BlockSpec((blk, 16), lambda i: (i, 0)) for _ in range(n_rels)]
        + [pl.BlockSpec((d_in, HID), lambda i: (0, 0)) for _ in range(n_rels)]
        + [pl.BlockSpec((blk, 128), lambda i: (i, 0)) for _ in range(npc)]
        + [
            pl.BlockSpec((d_in, HID), lambda i: (0, 0)),
            pl.BlockSpec((1, HID), lambda i: (0, 0)),
        ]
    )
    return pl.pallas_call(
        functools.partial(_z_body, n_rels, npc, nblk),
        grid=(nblk,),
        in_specs=in_specs,
        out_specs=[
            pl.BlockSpec((blk, HID), lambda i: (i, 0)),
            pl.BlockSpec((8, HID), lambda i: (0, 0)),
        ],
        out_shape=[
            jax.ShapeDtypeStruct((n, HID), jnp.float32),
            jax.ShapeDtypeStruct((8, HID), jnp.float32),
        ],
    )


def _bn_body(n, kind, z_ref, st_ref, g_ref, b_ref, *out_refs):
    inv_n = 1.0 / n
    m = st_ref[0:1, :] * inv_n
    var = st_ref[1:2, :] * inv_n - m * m
    scale = lax.rsqrt(var + 1e-5) * g_ref[...]
    y = jnp.maximum((z_ref[...] - m) * scale + b_ref[...], 0.0)
    if kind == "full":
        out_refs[0][...] = y
    else:
        for p in range(HID // 128):
            out_refs[p][...] = y[:, p * 128:(p + 1) * 128]
        if kind == "pieces+chunks":
            for c in range(HID // 32):
                out_refs[HID // 128 + c][...] = y[:, c * 32:(c + 1) * 32]


@functools.lru_cache(maxsize=None)
def _make_bn(n, kind, blk):
    nblk = n // blk
    out_specs, out_shape = [], []
    if kind == "full":
        out_specs.append(pl.BlockSpec((blk, HID), lambda i: (i, 0)))
        out_shape.append(jax.ShapeDtypeStruct((n, HID), jnp.float32))
    else:
        for _ in range(HID // 128):
            out_specs.append(pl.BlockSpec((blk, 128), lambda i: (i, 0)))
            out_shape.append(jax.ShapeDtypeStruct((n, 128), jnp.float32))
        if kind == "pieces+chunks":
            for _ in range(HID // 32):
                out_specs.append(pl.BlockSpec((blk, 32), lambda i: (i, 0)))
                out_shape.append(jax.ShapeDtypeStruct((n, 32), jnp.float32))
    return pl.pallas_call(
        functools.partial(_bn_body, n, kind),
        grid=(nblk,),
        in_specs=[
            pl.BlockSpec((blk, HID), lambda i: (i, 0)),
            pl.BlockSpec((8, HID), lambda i: (0, 0)),
            pl.BlockSpec((1, HID), lambda i: (0, 0)),
            pl.BlockSpec((1, HID), lambda i: (0, 0)),
        ],
        out_specs=out_specs,
        out_shape=out_shape,
    )


def _pad_edges(e, n_dst):
    e = e.astype(jnp.int32)
    src = jnp.concatenate([e[0], jnp.zeros((EP - E,), jnp.int32)])
    dst = jnp.concatenate([e[1], jnp.full((EP - E,), n_dst, jnp.int32)])
    return src.reshape(EP // T, T), dst.reshape(EP // T, T)


def _zeros(w):
    return jnp.zeros((ZROWS, w), jnp.float32)


def _layer(xpieces, x32, edges, counts, params, layer, blk=2000):
    d_in = 128 * len(xpieces[NTYPES[0]])
    aggs = {}
    for (cw, nbuf, work) in SC_PLAN:
        args = []
        for core in work:
            for (rel, half) in core:
                s_nt, _ = EDGE_BY_REL[rel]
                tables = x32[s_nt] if cw == 32 else xpieces[s_nt]
                args += list(tables) + list(edges[rel])
        outs = _make_agg_layer(d_in, cw, nbuf, work)(*args)
        oi = 0
        for core in work:
            for (rel, half) in core:
                key = rel if half is None else "%s_%d" % (rel, half)
                aggs[key] = outs[oi]
                oi += 1
    out = {}
    in_rels = {nt: [] for nt in NTYPES}
    for (s, rel, d) in ETYPES:
        if rel == "in_pathway":
            in_rels[d] += [("in_pathway_0", rel), ("in_pathway_1", rel)]
        else:
            in_rels[d].append((rel, rel))
    for nt in NTYPES:
        rels = in_rels[nt]
        n = NNODES[nt]
        wl_list = [params["W%dl_%s" % (layer, r)].T for (_, r) in rels]
        real = sorted({r for (_, r) in rels})
        wr = sum(params["W%dr_%s" % (layer, r)] for r in real).T
        bias = sum(params["b%dl_%s" % (layer, r)] for r in real).reshape(1, HID)
        z, st = _make_z(n, d_in, len(rels), blk)(
            *[aggs[k] for (k, _) in rels], *[counts[r] for (_, r) in rels],
            *wl_list, *xpieces[nt], wr, bias)
        out[nt] = (z, st)
    return out


def kernel(x_drug, x_protein, x_pathway, x_side_effect, ei_treats, ei_targets, ei_in_pathway, ei_causes, ei_rev_treats, ei_rev_targets, ei_rev_in_pathway, W1l_treats, b1l_treats, W1r_treats, W2l_treats, b2l_treats, W2r_treats, W1l_targets, b1l_targets, W1r_targets, W2l_targets, b2l_targets, W2r_targets, W1l_in_pathway, b1l_in_pathway, W1r_in_pathway, W2l_in_pathway, b2l_in_pathway, W2r_in_pathway, W1l_causes, b1l_causes, W1r_causes, W2l_causes, b2l_causes, W2r_causes, W1l_rev_treats, b1l_rev_treats, W1r_rev_treats, W2l_rev_treats, b2l_rev_treats, W2r_rev_treats, W1l_rev_targets, b1l_rev_targets, W1r_rev_targets, W2l_rev_targets, b2l_rev_targets, W2r_rev_targets, W1l_rev_in_pathway, b1l_rev_in_pathway, W1r_rev_in_pathway, W2l_rev_in_pathway, b2l_rev_in_pathway, W2r_rev_in_pathway, bn1_g, bn1_b, bn2_g, bn2_b):
    params = dict(locals())
    xd = {nt: params["x_" + nt] for nt in NTYPES}

    ones16 = jnp.ones((T, 16), jnp.float32)
    edges = {}
    for (s, rel, d) in ETYPES:
        edges[rel] = _pad_edges(params["ei_" + rel], NNODES[d])

    cnt_list = _make_counts()(*[edges[rel][1] for (_, rel, _) in ETYPES],
                              ones16, _zeros(16))
    counts = {rel: cnt_list[t] for t, (_, rel, _) in enumerate(ETYPES)}

    xpieces = {nt: [xd[nt]] for nt in NTYPES}
    x32 = {nt: [xd[nt][:, c * 32:(c + 1) * 32] for c in range(D_IN // 32)]
           for nt in CHUNK32_SRCS}
    z1 = _layer(xpieces, x32, edges, counts, params, 1)
    xp1, x32_1 = {}, {}
    for nt in NTYPES:
        z, st = z1[nt]
        kind = "pieces+chunks" if nt in CHUNK32_SRCS else "pieces"
        outs = _make_bn(NNODES[nt], kind, 1000)(
            z, st, bn1_g.reshape(1, HID), bn1_b.reshape(1, HID))
        xp1[nt] = outs[:HID // 128]
        if nt in CHUNK32_SRCS:
            x32_1[nt] = outs[HID // 128:]

    z2 = _layer(xp1, x32_1, edges, counts, params, 2)
    res = []
    for nt in NTYPES:
        z, st = z2[nt]
        outs = _make_bn(NNODES[nt], "full", 1000)(
            z, st, bn2_g.reshape(1, HID), bn2_b.reshape(1, HID))
        res.append(outs[0])
    return tuple(res)


# speedup vs baseline: 1.0178x; 1.0178x over previous
import functools

import jax
import jax.numpy as jnp
from jax import lax
from jax.experimental import pallas as pl
from jax.experimental.pallas import tpu as pltpu
from jax.experimental.pallas import tpu_sc as plsc

NTYPES = ["drug", "protein", "pathway", "side_effect"]
NNODES = {"drug": 10000, "protein": 50000, "pathway": 10000, "side_effect": 10000}
ETYPES = [
    ("drug", "treats", "side_effect"),
    ("drug", "targets", "protein"),
    ("protein", "in_pathway", "pathway"),
    ("protein", "causes", "side_effect"),
    ("side_effect", "rev_treats", "drug"),
    ("protein", "rev_targets", "drug"),
    ("pathway", "rev_in_pathway", "protein"),
]
D_IN, HID = 128, 256
E = 80000

NS = 16
T = 128
NT = 40
EP = NS * T * NT
NBUF = 4

SC_PLAN = [
    (32, 4, ((("targets", None),), (("rev_in_pathway", None),))),
    (128, 2, ((("treats", None), ("rev_treats", None)),
              (("causes", None), ("rev_targets", None)))),
    (128, 2, ((("in_pathway", 0),), (("in_pathway", 1),))),
]
EDGE_BY_REL = {rel: (s, d) for (s, rel, d) in ETYPES}
RELS = [rel for (_, rel, _) in ETYPES]
CHUNK32_SRCS = ("drug", "pathway")


def _pad128(n):
    return ((n + 128) // 128) * 128


ZROWS = _pad128(50000) // NS

_MESH = plsc.VectorSubcoreMesh(core_axis_name="c", subcore_axis_name="s")
_SC_PARAMS = pltpu.CompilerParams(use_tc_tiling_on_sc=False)


def _agg_type(cw, nbuf, nch, n_dst, nt_rows, row0, tables, src_hbm,
              dst_hbm, zbuf, out_hbm, acc, src_v, dst_v, rowbufs, gsems, sid):
    n_pad = _pad128(n_dst)
    rows_t = n_pad // NS
    zr = 4096 // cw
    base = row0 + sid * nt_rows
    pltpu.sync_copy(src_hbm.at[pl.ds(base, nt_rows)],
                    src_v.at[pl.ds(0, nt_rows)])
    pltpu.sync_copy(dst_hbm.at[pl.ds(base, nt_rows)],
                    dst_v.at[pl.ds(0, nt_rows)])
    for ch in range(nch):
        tcol = tables[ch]
        zoff = sid * rows_t

        def zstep(k, carry):
            pltpu.sync_copy(zbuf, acc.at[pl.ds(zoff + k * zr, zr)])
            return carry

        lax.fori_loop(0, rows_t // zr, zstep, 0)
        rem = rows_t % zr
        if rem:
            pltpu.sync_copy(
                zbuf.at[pl.ds(0, rem)],
                acc.at[pl.ds(zoff + (rows_t // zr) * zr, rem)])
        plsc.subcore_barrier()

        for b in range(nbuf):
            pltpu.async_copy(tcol.at[src_v.at[b]], rowbufs.at[b], gsems[b])

        def outer(g, carry, tcol=tcol):
            for b in range(nbuf):
                j = g * nbuf + b
                pltpu.make_async_copy(
                    tcol.at[src_v.at[j]], rowbufs.at[b], gsems[b]).wait()
                pltpu.sync_copy(rowbufs.at[b], acc.at[dst_v.at[j]], add=True)

                @pl.when(j + nbuf < nt_rows)
                def _():
                    pltpu.async_copy(tcol.at[src_v.at[j + nbuf]],
                                     rowbufs.at[b], gsems[b])
            return carry

        lax.fori_loop(0, nt_rows // nbuf, outer, 0)
        plsc.subcore_barrier()
        pltpu.sync_copy(acc.at[pl.ds(sid * rows_t, rows_t)],
                        out_hbm.at[pl.ds(sid * rows_t, rows_t),
                                   pl.ds(ch * cw, cw)])
        plsc.subcore_barrier()


@functools.lru_cache(maxsize=None)
def _make_agg_layer(d_in, cw, nbuf, work):
    nch = d_in // cw
    ents = [e for core in work for e in core]
    n_outs = [NNODES[EDGE_BY_REL[rel][1]] for (rel, _) in ents]
    acc_rows = _pad128(max(n_outs))

    def body(*refs):
        p = 0
        tabs, eidx = [], []
        for _ in ents:
            tabs.append(refs[p:p + nch])
            eidx.append((refs[p + nch], refs[p + nch + 1]))
            p += nch + 2
        nout = len(ents)
        outs = refs[p:p + nout]
        p += nout
        acc = refs[p]
        src_v, dst_v, rowbufs, zbuf = refs[p + 1:p + 5]
        gsems = refs[p + 5:]
        cid = lax.axis_index("c")
        sid = lax.axis_index("s")

        def zfill(i, carry):
            for w in range(cw // 16):
                zbuf[i, pl.ds(w * 16, 16)] = jnp.zeros((16,), jnp.float32)
            return carry

        lax.fori_loop(0, 4096 // cw, zfill, 0)

        def run(core):
            oi = 0 if core == 0 else len(work[0])
            for (rel, half) in work[core]:
                if half is None:
                    nt_rows, row0 = NT, 0
                else:
                    nt_rows, row0 = NT // 2, half * (EP // T // 2)
                _agg_type(cw, nbuf, nch, n_outs[oi], nt_rows, row0, tabs[oi],
                          eidx[oi][0], eidx[oi][1], zbuf, outs[oi],
                          acc, src_v, dst_v, rowbufs, gsems, sid)
                oi += 1

        @pl.when(cid == 0)
        def _():
            run(0)

        @pl.when(cid == 1)
        def _():
            run(1)

    return pl.kernel(
        body,
        out_type=[jax.ShapeDtypeStruct((_pad128(n), d_in), jnp.float32)
                  for n in n_outs],
        mesh=_MESH,
        compiler_params=_SC_PARAMS,
        scratch_types=[
            pltpu.VMEM_SHARED((acc_rows, cw), jnp.float32),
            pltpu.VMEM((NT, T), jnp.int32),
            pltpu.VMEM((NT, T), jnp.int32),
            pltpu.VMEM((nbuf, T, cw), jnp.float32),
            pltpu.VMEM((4096 // cw, cw), jnp.float32),
        ] + [pltpu.SemaphoreType.DMA] * nbuf,
    )


@functools.lru_cache(maxsize=None)
def _make_counts():
    n_dsts = tuple(NNODES[d] for (_, _, d) in ETYPES)

    def body(*refs):
        dsts = refs[:7]
        ones_hbm, zeros_hbm = refs[7:9]
        outs = refs[9:16]
        acc, dst_v, onesbuf = refs[16:]
        cid = lax.axis_index("c")
        sid = lax.axis_index("s")
        base = sid * NT
        pltpu.sync_copy(ones_hbm, onesbuf)

        def one_type(t):
            rows_t = _pad128(n_dsts[t]) // NS
            pltpu.sync_copy(zeros_hbm.at[pl.ds(0, rows_t)],
                            acc.at[pl.ds(sid * rows_t, rows_t)])
            pltpu.sync_copy(dsts[t].at[pl.ds(base, NT)], dst_v)
            plsc.subcore_barrier()

            def step(j, carry):
                pltpu.sync_copy(onesbuf, acc.at[dst_v.at[j]], add=True)
                return carry

            lax.fori_loop(0, NT, step, 0)
            plsc.subcore_barrier()
            pltpu.sync_copy(acc.at[pl.ds(sid * rows_t, rows_t)],
                            outs[t].at[pl.ds(sid * rows_t, rows_t)])
            plsc.subcore_barrier()

        @pl.when(cid == 0)
        def _():
            for t in (0, 1, 2, 3):
                one_type(t)

        @pl.when(cid == 1)
        def _():
            for t in (4, 5, 6):
                one_type(t)

    return pl.kernel(
        body,
        out_type=[jax.ShapeDtypeStruct((_pad128(n), 16), jnp.float32)
                  for n in n_dsts],
        mesh=_MESH,
        compiler_params=_SC_PARAMS,
        scratch_types=[
            pltpu.VMEM_SHARED((_pad128(50000), 16), jnp.float32),
            pltpu.VMEM((NT, T), jnp.int32),
            pltpu.VMEM((T, 16), jnp.float32),
        ],
    )


def _z_body(n_rels, npc, nblk, *refs):
    i = pl.program_id(0)
    aggs = refs[0:n_rels]
    cnts = refs[n_rels:2 * n_rels]
    wls = refs[2 * n_rels:3 * n_rels]
    xps = refs[3 * n_rels:3 * n_rels + npc]
    wr_ref, bias_ref, z_ref, st_ref = refs[3 * n_rels + npc:]
    z = bias_ref[...]
    for p in range(npc):
        z = z + jnp.dot(xps[p][...], wr_ref[pl.ds(p * 128, 128), :],
                        preferred_element_type=jnp.float32)
    for r in range(n_rels):
        a = aggs[r][...]
        cnt = cnts[r][:, 0:1]
        mean = a * (1.0 / jnp.maximum(cnt, 1.0))
        z = z + jnp.dot(mean, wls[r][...], preferred_element_type=jnp.float32)
    z_ref[...] = z

    @pl.when(i == 0)
    def _():
        st_ref[...] = jnp.zeros_like(st_ref)

    st_ref[0:1, :] += jnp.sum(z, axis=0, keepdims=True)
    st_ref[1:2, :] += jnp.sum(z * z, axis=0, keepdims=True)


@functools.lru_cache(maxsize=None)
def _make_z(n, d_in, n_rels, blk):
    nblk = n // blk
    npc = d_in // 128
    in_specs = (
        [pl.BlockSpec((blk, d_in), lambda i: (i, 0))
         for _ in range(n_rels)]
        + [pl.BlockSpec((blk, 16), lambda i: (i, 0)) for _ in range(n_rels)]
        + [pl.BlockSpec((d_in, HID), lambda i: (0, 0)) for _ in range(n_rels)]
        + [pl.BlockSpec((blk, 128), lambda i: (i, 0)) for _ in range(npc)]
        + [
            pl.BlockSpec((d_in, HID), lambda i: (0, 0)),
            pl.BlockSpec((1, HID), lambda i: (0, 0)),
        ]
    )
    return pl.pallas_call(
        functools.partial(_z_body, n_rels, npc, nblk),
        grid=(nblk,),
        in_specs=in_specs,
        out_specs=[
            pl.BlockSpec((blk, HID), lambda i: (i, 0)),
            pl.BlockSpec((8, HID), lambda i: (0, 0)),
        ],
        out_shape=[
            jax.ShapeDtypeStruct((n, HID), jnp.float32),
            jax.ShapeDtypeStruct((8, HID), jnp.float32),
        ],
    )


def _bn_body(n, kind, z_ref, st_ref, g_ref, b_ref, *out_refs):
    inv_n = 1.0 / n
    m = st_ref[0:1, :] * inv_n
    var = st_ref[1:2, :] * inv_n - m * m
    scale = lax.rsqrt(var + 1e-5) * g_ref[...]
    y = jnp.maximum((z_ref[...] - m) * scale + b_ref[...], 0.0)
    if kind == "full":
        out_refs[0][...] = y
    else:
        for p in range(HID // 128):
            out_refs[p][...] = y[:, p * 128:(p + 1) * 128]
        if kind == "pieces+chunks":
            for c in range(HID // 32):
                out_refs[HID // 128 + c][...] = y[:, c * 32:(c + 1) * 32]


@functools.lru_cache(maxsize=None)
def _make_bn(n, kind, blk):
    nblk = n // blk
    out_specs, out_shape = [], []
    if kind == "full":
        out_specs.append(pl.BlockSpec((blk, HID), lambda i: (i, 0)))
        out_shape.append(jax.ShapeDtypeStruct((n, HID), jnp.float32))
    else:
        for _ in range(HID // 128):
            out_specs.append(pl.BlockSpec((blk, 128), lambda i: (i, 0)))
            out_shape.append(jax.ShapeDtypeStruct((n, 128), jnp.float32))
        if kind == "pieces+chunks":
            for _ in range(HID // 32):
                out_specs.append(pl.BlockSpec((blk, 32), lambda i: (i, 0)))
                out_shape.append(jax.ShapeDtypeStruct((n, 32), jnp.float32))
    return pl.pallas_call(
        functools.partial(_bn_body, n, kind),
        grid=(nblk,),
        in_specs=[
            pl.BlockSpec((blk, HID), lambda i: (i, 0)),
            pl.BlockSpec((8, HID), lambda i: (0, 0)),
            pl.BlockSpec((1, HID), lambda i: (0, 0)),
            pl.BlockSpec((1, HID), lambda i: (0, 0)),
        ],
        out_specs=out_specs,
        out_shape=out_shape,
    )


def _pad_edges(e, n_dst):
    e = e.astype(jnp.int32)
    src = jnp.concatenate([e[0], jnp.zeros((EP - E,), jnp.int32)])
    dst = jnp.concatenate([e[1], jnp.full((EP - E,), n_dst, jnp.int32)])
    return src.reshape(EP // T, T), dst.reshape(EP // T, T)


def _zeros(w):
    return jnp.zeros((ZROWS, w), jnp.float32)


def _layer(xpieces, x32, edges, counts, params, layer, blk=1000):
    d_in = 128 * len(xpieces[NTYPES[0]])
    aggs = {}
    for (cw, nbuf, work) in SC_PLAN:
        args = []
        for core in work:
            for (rel, half) in core:
                s_nt, _ = EDGE_BY_REL[rel]
                tables = x32[s_nt] if cw == 32 else xpieces[s_nt]
                args += list(tables) + list(edges[rel])
        outs = _make_agg_layer(d_in, cw, nbuf, work)(*args)
        oi = 0
        for core in work:
            for (rel, half) in core:
                key = rel if half is None else "%s_%d" % (rel, half)
                aggs[key] = outs[oi]
                oi += 1
    out = {}
    in_rels = {nt: [] for nt in NTYPES}
    for (s, rel, d) in ETYPES:
        if rel == "in_pathway":
            in_rels[d] += [("in_pathway_0", rel), ("in_pathway_1", rel)]
        else:
            in_rels[d].append((rel, rel))
    for nt in NTYPES:
        rels = in_rels[nt]
        n = NNODES[nt]
        wl_list = [params["W%dl_%s" % (layer, r)].T for (_, r) in rels]
        real = sorted({r for (_, r) in rels})
        wr = sum(params["W%dr_%s" % (layer, r)] for r in real).T
        bias = sum(params["b%dl_%s" % (layer, r)] for r in real).reshape(1, HID)
        z, st = _make_z(n, d_in, len(rels), blk)(
            *[aggs[k] for (k, _) in rels], *[counts[r] for (_, r) in rels],
            *wl_list, *xpieces[nt], wr, bias)
        out[nt] = (z, st)
    return out


def kernel(x_drug, x_protein, x_pathway, x_side_effect, ei_treats, ei_targets, ei_in_pathway, ei_causes, ei_rev_treats, ei_rev_targets, ei_rev_in_pathway, W1l_treats, b1l_treats, W1r_treats, W2l_treats, b2l_treats, W2r_treats, W1l_targets, b1l_targets, W1r_targets, W2l_targets, b2l_targets, W2r_targets, W1l_in_pathway, b1l_in_pathway, W1r_in_pathway, W2l_in_pathway, b2l_in_pathway, W2r_in_pathway, W1l_causes, b1l_causes, W1r_causes, W2l_causes, b2l_causes, W2r_causes, W1l_rev_treats, b1l_rev_treats, W1r_rev_treats, W2l_rev_treats, b2l_rev_treats, W2r_rev_treats, W1l_rev_targets, b1l_rev_targets, W1r_rev_targets, W2l_rev_targets, b2l_rev_targets, W2r_rev_targets, W1l_rev_in_pathway, b1l_rev_in_pathway, W1r_rev_in_pathway, W2l_rev_in_pathway, b2l_rev_in_pathway, W2r_rev_in_pathway, bn1_g, bn1_b, bn2_g, bn2_b):
    params = dict(locals())
    xd = {nt: params["x_" + nt] for nt in NTYPES}

    ones16 = jnp.ones((T, 16), jnp.float32)
    edges = {}
    for (s, rel, d) in ETYPES:
        edges[rel] = _pad_edges(params["ei_" + rel], NNODES[d])

    cnt_list = _make_counts()(*[edges[rel][1] for (_, rel, _) in ETYPES],
                              ones16, _zeros(16))
    counts = {rel: cnt_list[t] for t, (_, rel, _) in enumerate(ETYPES)}

    xpieces = {nt: [xd[nt]] for nt in NTYPES}
    x32 = {nt: [xd[nt][:, c * 32:(c + 1) * 32] for c in range(D_IN // 32)]
           for nt in CHUNK32_SRCS}
    z1 = _layer(xpieces, x32, edges, counts, params, 1)
    xp1, x32_1 = {}, {}
    for nt in NTYPES:
        z, st = z1[nt]
        kind = "pieces+chunks" if nt in CHUNK32_SRCS else "pieces"
        outs = _make_bn(NNODES[nt], kind, 1000)(
            z, st, bn1_g.reshape(1, HID), bn1_b.reshape(1, HID))
        xp1[nt] = outs[:HID // 128]
        if nt in CHUNK32_SRCS:
            x32_1[nt] = outs[HID // 128:]

    z2 = _layer(xp1, x32_1, edges, counts, params, 2)
    res = []
    for nt in NTYPES:
        z, st = z2[nt]
        outs = _make_bn(NNODES[nt], "full", 1000)(
            z, st, bn2_g.reshape(1, HID), bn2_b.reshape(1, HID))
        res.append(outs[0])
    return tuple(res)
